# Initial kernel scaffold; baseline (speedup 1.0000x reference)
import jax
import jax.numpy as jnp
from jax.experimental import pallas as pl


def kernel(features, x_atom, edge_attr, edge_index, W1, b1, g1, be1, W2, b2, g2, be2):
    raise NotImplementedError("write your pallas kernel here")


# speedup vs baseline: 2.9012x; 2.9012x over previous
import functools

import jax
import jax.numpy as jnp
from jax import lax
from jax.experimental import pallas as pl
from jax.experimental.pallas import tpu as pltpu
from jax.experimental.pallas import tpu_sc as plsc

NC = 2
NS = 16
NW = NC * NS
L = 16
C = 128
EPS = 1e-5


def _mesh():
    return plsc.VectorSubcoreMesh(
        core_axis_name="c", subcore_axis_name="s", num_cores=NC, num_subcores=NS
    )


def _a1_body(f_ref, x_ref, wh_ref, wx_ref, o_ref):
    o_ref[...] = (
        jnp.dot(f_ref[...], wh_ref[...], preferred_element_type=jnp.float32)
        + jnp.dot(x_ref[...], wx_ref[...], preferred_element_type=jnp.float32)
    )


def _a2_body(e_ref, w_ref, b_ref, o_ref):
    o_ref[...] = (
        jnp.dot(e_ref[...], w_ref[...], preferred_element_type=jnp.float32)
        + b_ref[...]
    )


def _e1_body(nvalid, bn_e, c0_ref, c1_ref, w2_ref, b2_ref, y_ref, st_ref):
    i = pl.program_id(0)
    y = (
        jnp.dot(c0_ref[...] + c1_ref[...], w2_ref[...],
                preferred_element_type=jnp.float32)
        + b2_ref[...]
    )
    y_ref[...] = y
    rows = lax.broadcasted_iota(jnp.int32, (bn_e, 1), 0) + i * bn_e
    ym = jnp.where(rows < nvalid, y, 0.0)
    st_ref[0, 0, :] = jnp.sum(ym, axis=0)
    st_ref[0, 1, :] = jnp.sum(ym * ym, axis=0)


def _e2_body(y_ref, f_ref, st_ref, o_ref):
    st = st_ref[...]
    o_ref[...] = jnp.maximum(y_ref[...] * st[0:1, :] + st[1:2, :] + f_ref[...], 0.0)


def _make_pass_a(nchunk, n_nodes, h):
    nj = h // L

    @functools.partial(
        pl.kernel,
        out_type=jax.ShapeDtypeStruct((NW * 2 * h,), jnp.float32),
        mesh=_mesh(),
        scratch_types=[
            pltpu.VMEM((C,), jnp.int32),
            pltpu.VMEM((C, h), jnp.float32),
            pltpu.VMEM((C, h), jnp.float32),
            pltpu.VMEM((2 * h,), jnp.float32),
            pltpu.SemaphoreType.DMA,
        ],
    )
    def pass_a(src_hbm, p_hbm, q_hbm, out_hbm, idx_v, rows_v, q_v, st_v, sem):
        wid = lax.axis_index("s") * NC + lax.axis_index("c")
        c_lo = wid * nchunk // NW
        c_hi = (wid + 1) * nchunk // NW

        def chunk_body(c, carry):
            pltpu.sync_copy(src_hbm.at[pl.ds(c * C, C)], idx_v)
            pltpu.async_copy(p_hbm.at[idx_v], rows_v, sem).wait()
            pltpu.sync_copy(q_hbm.at[c], q_v)

            def row_body(r, accs):
                out = list(accs)
                for j in range(nj):
                    sl = pl.ds(j * L, L)
                    m = rows_v[r, sl] + q_v[r, sl]
                    out[j] = accs[j] + m
                    out[nj + j] = accs[nj + j] + m * m
                return tuple(out)

            return lax.fori_loop(0, C, row_body, carry)

        zero = jnp.zeros((L,), jnp.float32)
        carry = tuple(zero for _ in range(2 * nj))
        carry = lax.fori_loop(c_lo, c_hi, chunk_body, carry)
        for j in range(2 * nj):
            st_v[pl.ds(j * L, L)] = carry[j]
        pltpu.sync_copy(st_v, out_hbm.at[pl.ds(wid * 2 * h, 2 * h)])

    return pass_a


def _make_pass_b(nchunk, n_nodes, h):
    nj = h // L
    acc_rows = 10112
    tile_rows = acc_rows // NS
    chunks = [(0, C), (C, C), (2 * C, C), (3 * C, C), (4 * C, tile_rows - 4 * C)]
    assert acc_rows >= n_nodes and tile_rows % 8 == 0

    @functools.partial(
        pl.kernel,
        out_type=jax.ShapeDtypeStruct((NC, acc_rows, h), jnp.float32),
        mesh=_mesh(),
        scratch_types=[
            pltpu.VMEM((C,), jnp.int32),
            pltpu.VMEM((C,), jnp.int32),
            pltpu.VMEM((C, h), jnp.float32),
            pltpu.VMEM((C, h), jnp.float32),
            pltpu.VMEM((C, h), jnp.float32),
            pltpu.VMEM((2, h), jnp.float32),
            pltpu.VMEM_SHARED((acc_rows, h), jnp.float32),
            pltpu.SemaphoreType.DMA,
        ],
    )
    def pass_b(src_hbm, dst_hbm, p_hbm, f_hbm, q_hbm, st_hbm, out_hbm,
               idx_s, idx_d, rows_p, rows_f, q_v, st_v, acc, sem):
        sc = lax.axis_index("c")
        tid = lax.axis_index("s")
        wid = tid * NC + sc
        c_lo = wid * nchunk // NW
        c_hi = (wid + 1) * nchunk // NW

        def zrow(r, _):
            for j in range(nj):
                rows_p[r, pl.ds(j * L, L)] = jnp.zeros((L,), jnp.float32)
            return 0

        lax.fori_loop(0, C, zrow, 0)
        for off, sz in chunks:
            pltpu.sync_copy(rows_p.at[pl.ds(0, sz)],
                            acc.at[pl.ds(tid * tile_rows + off, sz)])
        plsc.subcore_barrier()

        pltpu.sync_copy(st_hbm, st_v)
        s1 = [st_v[0, pl.ds(j * L, L)] for j in range(nj)]
        t1 = [st_v[1, pl.ds(j * L, L)] for j in range(nj)]

        def chunk_body(c, _):
            pltpu.sync_copy(src_hbm.at[pl.ds(c * C, C)], idx_s)
            pltpu.sync_copy(dst_hbm.at[pl.ds(c * C, C)], idx_d)
            cp_p = pltpu.async_copy(p_hbm.at[idx_s], rows_p, sem)
            cp_f = pltpu.async_copy(f_hbm.at[idx_s], rows_f, sem)
            pltpu.sync_copy(q_hbm.at[c], q_v)
            cp_p.wait()
            cp_f.wait()

            def row_body(r, _):
                for j in range(nj):
                    sl = pl.ds(j * L, L)
                    m = rows_p[r, sl] + q_v[r, sl]
                    y = m * s1[j] + t1[j] + rows_f[r, sl]
                    q_v[r, sl] = jnp.maximum(y, 0.0)
                return 0

            lax.fori_loop(0, C, row_body, 0)
            pltpu.sync_copy(q_v, acc.at[idx_d], add=True)
            return 0

        lax.fori_loop(c_lo, c_hi, chunk_body, 0)
        plsc.subcore_barrier()

        for off, sz in chunks:
            r0 = tid * tile_rows + off
            pltpu.sync_copy(acc.at[pl.ds(r0, sz)], rows_p.at[pl.ds(0, sz)])
            pltpu.sync_copy(rows_p.at[pl.ds(0, sz)], out_hbm.at[sc, pl.ds(r0, sz)])

    return pass_b


def kernel(features, x_atom, edge_attr, edge_index, W1, b1, g1, be1, W2, b2, g2, be2):
    n, h = features.shape
    a = x_atom.shape[1]
    e, bf = edge_attr.shape
    nchunk = e // C
    assert nchunk * C == e

    src = edge_index[0]
    dst = edge_index[1]
    w1h = W1[:h]
    w1x = W1[h:h + a]
    w1w = W1[h + a:]

    bn = 1000
    p = pl.pallas_call(
        _a1_body,
        grid=(n // bn,),
        in_specs=[
            pl.BlockSpec((bn, h), lambda i: (i, 0)),
            pl.BlockSpec((bn, a), lambda i: (i, 0)),
            pl.BlockSpec((h, h), lambda i: (0, 0)),
            pl.BlockSpec((a, h), lambda i: (0, 0)),
        ],
        out_specs=pl.BlockSpec((bn, h), lambda i: (i, 0)),
        out_shape=jax.ShapeDtypeStruct((n, h), jnp.float32),
    )(features, x_atom, w1h, w1x)

    be_ = 4000
    q = pl.pallas_call(
        _a2_body,
        grid=(e // be_,),
        in_specs=[
            pl.BlockSpec((be_, bf), lambda i: (i, 0)),
            pl.BlockSpec((bf, h), lambda i: (0, 0)),
            pl.BlockSpec((1, h), lambda i: (0, 0)),
        ],
        out_specs=pl.BlockSpec((be_, h), lambda i: (i, 0)),
        out_shape=jax.ShapeDtypeStruct((e, h), jnp.float32),
    )(edge_attr, w1w, b1.reshape(1, h))
    q3 = q.reshape(nchunk, C, h)

    parts = _make_pass_a(nchunk, n, h)(src, p, q3).reshape(NW, 2 * h)
    sums = jnp.sum(parts[:, :h], axis=0)
    sqs = jnp.sum(parts[:, h:], axis=0)
    mu1 = sums / e
    var1 = sqs / e - mu1 * mu1
    s1 = g1 / jnp.sqrt(var1 + EPS)
    t1 = be1 - mu1 * s1
    st1 = jnp.stack([s1, t1])

    acc_rows = 10112
    hsc = _make_pass_b(nchunk, n, h)(src, dst, p, features, q3, st1)
    h2 = hsc.reshape(NC * acc_rows, h)

    bn_e = 632
    ng_e = acc_rows // bn_e
    y, st_parts = pl.pallas_call(
        functools.partial(_e1_body, n, bn_e),
        grid=(ng_e,),
        in_specs=[
            pl.BlockSpec((bn_e, h), lambda i: (i, 0)),
            pl.BlockSpec((bn_e, h), lambda i: (i + 16, 0)),
            pl.BlockSpec((h, h), lambda i: (0, 0)),
            pl.BlockSpec((1, h), lambda i: (0, 0)),
        ],
        out_specs=[
            pl.BlockSpec((bn_e, h), lambda i: (i, 0)),
            pl.BlockSpec((1, 2, h), lambda i: (i, 0, 0)),
        ],
        out_shape=[
            jax.ShapeDtypeStruct((acc_rows, h), jnp.float32),
            jax.ShapeDtypeStruct((ng_e, 2, h), jnp.float32),
        ],
    )(h2, h2, W2, b2.reshape(1, h))

    mu2 = jnp.sum(st_parts[:, 0, :], axis=0) / n
    var2 = jnp.sum(st_parts[:, 1, :], axis=0) / n - mu2 * mu2
    s2 = g2 / jnp.sqrt(var2 + EPS)
    t2 = be2 - mu2 * s2
    st2 = jnp.stack([s2, t2])

    out = pl.pallas_call(
        _e2_body,
        grid=(n // bn,),
        in_specs=[
            pl.BlockSpec((bn, h), lambda i: (i, 0)),
            pl.BlockSpec((bn, h), lambda i: (i, 0)),
            pl.BlockSpec((2, h), lambda i: (0, 0)),
        ],
        out_specs=pl.BlockSpec((bn, h), lambda i: (i, 0)),
        out_shape=jax.ShapeDtypeStruct((n, h), jnp.float32),
    )(y, features, st2)
    return out


# speedup vs baseline: 3.2083x; 1.1059x over previous
import functools

import jax
import jax.numpy as jnp
from jax import lax
from jax.experimental import pallas as pl
from jax.experimental.pallas import tpu as pltpu
from jax.experimental.pallas import tpu_sc as plsc

NC = 2
NS = 16
NW = NC * NS
L = 16
C = 128
EPS = 1e-5


def _mesh():
    return plsc.VectorSubcoreMesh(
        core_axis_name="c", subcore_axis_name="s", num_cores=NC, num_subcores=NS
    )


def _a1_body(f_ref, x_ref, wh_ref, wx_ref, o_ref):
    o_ref[...] = (
        jnp.dot(f_ref[...], wh_ref[...], preferred_element_type=jnp.float32)
        + jnp.dot(x_ref[...], wx_ref[...], preferred_element_type=jnp.float32)
    )


def _a2_body(e_ref, w_ref, b_ref, o_ref):
    o_ref[...] = (
        jnp.dot(e_ref[...], w_ref[...], preferred_element_type=jnp.float32)
        + b_ref[...]
    )


def _e1_body(nvalid, bn_e, c0_ref, c1_ref, w2_ref, b2_ref, y_ref, st_ref):
    i = pl.program_id(0)
    y = (
        jnp.dot(c0_ref[...] + c1_ref[...], w2_ref[...],
                preferred_element_type=jnp.float32)
        + b2_ref[...]
    )
    y_ref[...] = y
    rows = lax.broadcasted_iota(jnp.int32, (bn_e, 1), 0) + i * bn_e
    ym = jnp.where(rows < nvalid, y, 0.0)
    st_ref[0, 0, :] = jnp.sum(ym, axis=0)
    st_ref[0, 1, :] = jnp.sum(ym * ym, axis=0)


def _t_body(p_ref, f_ref, st_ref, o_ref):
    o_ref[...] = p_ref[...] * st_ref[0:1, :] + f_ref[...]


def _e2_body(y_ref, f_ref, st_ref, o_ref):
    st = st_ref[...]
    o_ref[...] = jnp.maximum(y_ref[...] * st[0:1, :] + st[1:2, :] + f_ref[...], 0.0)


def _make_pass_a(nchunk, n_nodes, h):
    nj = h // L

    @functools.partial(
        pl.kernel,
        out_type=jax.ShapeDtypeStruct((NW * 2 * h,), jnp.float32),
        mesh=_mesh(),
        scratch_types=[
            pltpu.VMEM((C,), jnp.int32),
            pltpu.VMEM((C, h), jnp.float32),
            pltpu.VMEM((C, h), jnp.float32),
            pltpu.VMEM((2 * h,), jnp.float32),
            pltpu.SemaphoreType.DMA,
        ],
    )
    def pass_a(src_hbm, p_hbm, q_hbm, out_hbm, idx_v, rows_v, q_v, st_v, sem):
        wid = lax.axis_index("s") * NC + lax.axis_index("c")
        c_lo = wid * nchunk // NW
        c_hi = (wid + 1) * nchunk // NW

        def chunk_body(c, carry):
            pltpu.sync_copy(src_hbm.at[pl.ds(c * C, C)], idx_v)
            pltpu.async_copy(p_hbm.at[idx_v], rows_v, sem).wait()
            pltpu.sync_copy(q_hbm.at[c], q_v)

            def row_body(r, accs):
                out = list(accs)
                for j in range(nj):
                    sl = pl.ds(j * L, L)
                    m = rows_v[r, sl] + q_v[r, sl]
                    out[j] = accs[j] + m
                    out[nj + j] = accs[nj + j] + m * m
                return tuple(out)

            return lax.fori_loop(0, C, row_body, carry)

        zero = jnp.zeros((L,), jnp.float32)
        carry = tuple(zero for _ in range(2 * nj))
        carry = lax.fori_loop(c_lo, c_hi, chunk_body, carry)
        for j in range(2 * nj):
            st_v[pl.ds(j * L, L)] = carry[j]
        pltpu.sync_copy(st_v, out_hbm.at[pl.ds(wid * 2 * h, 2 * h)])

    return pass_a


def _make_pass_b(nchunk, n_nodes, h):
    nj = h // L
    acc_rows = 10112
    tile_rows = acc_rows // NS
    chunks = [(0, C), (C, C), (2 * C, C), (3 * C, C), (4 * C, tile_rows - 4 * C)]
    assert acc_rows >= n_nodes and tile_rows % 8 == 0

    @functools.partial(
        pl.kernel,
        out_type=jax.ShapeDtypeStruct((NC, acc_rows, h), jnp.float32),
        mesh=_mesh(),
        scratch_types=[
            pltpu.VMEM((C,), jnp.int32),
            pltpu.VMEM((C,), jnp.int32),
            pltpu.VMEM((C, h), jnp.float32),
            pltpu.VMEM((C, h), jnp.float32),
            pltpu.VMEM((2, h), jnp.float32),
            pltpu.VMEM_SHARED((acc_rows, h), jnp.float32),
            pltpu.SemaphoreType.DMA,
        ],
    )
    def pass_b(src_hbm, dst_hbm, t_hbm, q_hbm, st_hbm, out_hbm,
               idx_s, idx_d, rows_p, q_v, st_v, acc, sem):
        sc = lax.axis_index("c")
        tid = lax.axis_index("s")
        wid = tid * NC + sc
        c_lo = wid * nchunk // NW
        c_hi = (wid + 1) * nchunk // NW

        def zrow(r, _):
            for j in range(nj):
                rows_p[r, pl.ds(j * L, L)] = jnp.zeros((L,), jnp.float32)
            return 0

        lax.fori_loop(0, C, zrow, 0)
        for off, sz in chunks:
            pltpu.sync_copy(rows_p.at[pl.ds(0, sz)],
                            acc.at[pl.ds(tid * tile_rows + off, sz)])
        plsc.subcore_barrier()

        pltpu.sync_copy(st_hbm, st_v)
        s1 = [st_v[0, pl.ds(j * L, L)] for j in range(nj)]
        t1 = [st_v[1, pl.ds(j * L, L)] for j in range(nj)]

        def chunk_body(c, _):
            pltpu.sync_copy(src_hbm.at[pl.ds(c * C, C)], idx_s)
            pltpu.sync_copy(dst_hbm.at[pl.ds(c * C, C)], idx_d)
            cp_t = pltpu.async_copy(t_hbm.at[idx_s], rows_p, sem)
            pltpu.sync_copy(q_hbm.at[c], q_v)
            cp_t.wait()

            def row_body(r, _):
                for j in range(nj):
                    sl = pl.ds(j * L, L)
                    y = q_v[r, sl] * s1[j] + t1[j] + rows_p[r, sl]
                    q_v[r, sl] = jnp.maximum(y, 0.0)
                return 0

            lax.fori_loop(0, C, row_body, 0)
            pltpu.sync_copy(q_v, acc.at[idx_d], add=True)
            return 0

        lax.fori_loop(c_lo, c_hi, chunk_body, 0)
        plsc.subcore_barrier()

        for off, sz in chunks:
            r0 = tid * tile_rows + off
            pltpu.sync_copy(acc.at[pl.ds(r0, sz)], rows_p.at[pl.ds(0, sz)])
            pltpu.sync_copy(rows_p.at[pl.ds(0, sz)], out_hbm.at[sc, pl.ds(r0, sz)])

    return pass_b


def kernel(features, x_atom, edge_attr, edge_index, W1, b1, g1, be1, W2, b2, g2, be2):
    n, h = features.shape
    a = x_atom.shape[1]
    e, bf = edge_attr.shape
    nchunk = e // C
    assert nchunk * C == e

    src = edge_index[0]
    dst = edge_index[1]
    w1h = W1[:h]
    w1x = W1[h:h + a]
    w1w = W1[h + a:]

    bn = 1000
    p = pl.pallas_call(
        _a1_body,
        grid=(n // bn,),
        in_specs=[
            pl.BlockSpec((bn, h), lambda i: (i, 0)),
            pl.BlockSpec((bn, a), lambda i: (i, 0)),
            pl.BlockSpec((h, h), lambda i: (0, 0)),
            pl.BlockSpec((a, h), lambda i: (0, 0)),
        ],
        out_specs=pl.BlockSpec((bn, h), lambda i: (i, 0)),
        out_shape=jax.ShapeDtypeStruct((n, h), jnp.float32),
    )(features, x_atom, w1h, w1x)

    be_ = 4000
    q = pl.pallas_call(
        _a2_body,
        grid=(e // be_,),
        in_specs=[
            pl.BlockSpec((be_, bf), lambda i: (i, 0)),
            pl.BlockSpec((bf, h), lambda i: (0, 0)),
            pl.BlockSpec((1, h), lambda i: (0, 0)),
        ],
        out_specs=pl.BlockSpec((be_, h), lambda i: (i, 0)),
        out_shape=jax.ShapeDtypeStruct((e, h), jnp.float32),
    )(edge_attr, w1w, b1.reshape(1, h))
    q3 = q.reshape(nchunk, C, h)

    parts = _make_pass_a(nchunk, n, h)(src, p, q3).reshape(NW, 2 * h)
    sums = jnp.sum(parts[:, :h], axis=0)
    sqs = jnp.sum(parts[:, h:], axis=0)
    mu1 = sums / e
    var1 = sqs / e - mu1 * mu1
    s1 = g1 / jnp.sqrt(var1 + EPS)
    t1 = be1 - mu1 * s1
    st1 = jnp.stack([s1, t1])

    t_tab = pl.pallas_call(
        _t_body,
        grid=(n // bn,),
        in_specs=[
            pl.BlockSpec((bn, h), lambda i: (i, 0)),
            pl.BlockSpec((bn, h), lambda i: (i, 0)),
            pl.BlockSpec((2, h), lambda i: (0, 0)),
        ],
        out_specs=pl.BlockSpec((bn, h), lambda i: (i, 0)),
        out_shape=jax.ShapeDtypeStruct((n, h), jnp.float32),
    )(p, features, st1)

    acc_rows = 10112
    hsc = _make_pass_b(nchunk, n, h)(src, dst, t_tab, q3, st1)
    h2 = hsc.reshape(NC * acc_rows, h)

    bn_e = 632
    ng_e = acc_rows // bn_e
    y, st_parts = pl.pallas_call(
        functools.partial(_e1_body, n, bn_e),
        grid=(ng_e,),
        in_specs=[
            pl.BlockSpec((bn_e, h), lambda i: (i, 0)),
            pl.BlockSpec((bn_e, h), lambda i: (i + 16, 0)),
            pl.BlockSpec((h, h), lambda i: (0, 0)),
            pl.BlockSpec((1, h), lambda i: (0, 0)),
        ],
        out_specs=[
            pl.BlockSpec((bn_e, h), lambda i: (i, 0)),
            pl.BlockSpec((1, 2, h), lambda i: (i, 0, 0)),
        ],
        out_shape=[
            jax.ShapeDtypeStruct((acc_rows, h), jnp.float32),
            jax.ShapeDtypeStruct((ng_e, 2, h), jnp.float32),
        ],
    )(h2, h2, W2, b2.reshape(1, h))

    mu2 = jnp.sum(st_parts[:, 0, :], axis=0) / n
    var2 = jnp.sum(st_parts[:, 1, :], axis=0) / n - mu2 * mu2
    s2 = g2 / jnp.sqrt(var2 + EPS)
    t2 = be2 - mu2 * s2
    st2 = jnp.stack([s2, t2])

    out = pl.pallas_call(
        _e2_body,
        grid=(n // bn,),
        in_specs=[
            pl.BlockSpec((bn, h), lambda i: (i, 0)),
            pl.BlockSpec((bn, h), lambda i: (i, 0)),
            pl.BlockSpec((2, h), lambda i: (0, 0)),
        ],
        out_specs=pl.BlockSpec((bn, h), lambda i: (i, 0)),
        out_shape=jax.ShapeDtypeStruct((n, h), jnp.float32),
    )(y, features, st2)
    return out


# speedup vs baseline: 5.1698x; 1.6114x over previous
import functools

import jax
import jax.numpy as jnp
from jax import lax
from jax.experimental import pallas as pl
from jax.experimental.pallas import tpu as pltpu
from jax.experimental.pallas import tpu_sc as plsc

NC = 2
NS = 16
NW = NC * NS
L = 16
C = 128
EPS = 1e-5


def _mesh():
    return plsc.VectorSubcoreMesh(
        core_axis_name="c", subcore_axis_name="s", num_cores=NC, num_subcores=NS
    )


def _a1_body(f_ref, x_ref, wh_ref, wx_ref, o_ref):
    o_ref[...] = (
        jnp.dot(f_ref[...], wh_ref[...], preferred_element_type=jnp.float32)
        + jnp.dot(x_ref[...], wx_ref[...], preferred_element_type=jnp.float32)
    )


def _a2_body(e_ref, w_ref, b_ref, o_ref):
    o_ref[...] = (
        jnp.dot(e_ref[...], w_ref[...], preferred_element_type=jnp.float32)
        + b_ref[...]
    )


def _t_body(p_ref, f_ref, st_ref, o_ref):
    o_ref[...] = p_ref[...] * st_ref[0:1, :] + f_ref[...]


def _e1_body(nvalid, bn_e, c0_ref, c1_ref, w2_ref, b2_ref, y_ref, st_ref):
    i = pl.program_id(0)
    y = (
        jnp.dot(c0_ref[...] + c1_ref[...], w2_ref[...],
                preferred_element_type=jnp.float32)
        + b2_ref[...]
    )
    y_ref[...] = y
    rows = lax.broadcasted_iota(jnp.int32, (bn_e, 1), 0) + i * bn_e
    ym = jnp.where(rows < nvalid, y, 0.0)
    st_ref[0, 0, :] = jnp.sum(ym, axis=0)
    st_ref[0, 1, :] = jnp.sum(ym * ym, axis=0)


def _e2_body(y_ref, f_ref, st_ref, o_ref):
    st = st_ref[...]
    o_ref[...] = jnp.maximum(y_ref[...] * st[0:1, :] + st[1:2, :] + f_ref[...], 0.0)


def _make_pass_a(e_edges, h):
    nj = h // L
    ew = e_edges // NW
    nch = ew // C
    tail = ew - nch * C
    assert ew % 8 == 0 and tail % 8 == 0 and nch >= 4 and nch % 2 == 0
    tl = max(tail, 8)

    @functools.partial(
        pl.kernel,
        out_type=jax.ShapeDtypeStruct((NW * 2 * h,), jnp.float32),
        mesh=_mesh(),
        scratch_types=[
            pltpu.VMEM((C,), jnp.int32),
            pltpu.VMEM((C,), jnp.int32),
            pltpu.VMEM((C, h), jnp.float32),
            pltpu.VMEM((C, h), jnp.float32),
            pltpu.VMEM((C, h), jnp.float32),
            pltpu.VMEM((C, h), jnp.float32),
            pltpu.VMEM((tl, h), jnp.float32),
            pltpu.VMEM((tl, h), jnp.float32),
            pltpu.VMEM((tl,), jnp.int32),
            pltpu.VMEM((2 * h,), jnp.float32),
            pltpu.SemaphoreType.DMA,
            pltpu.SemaphoreType.DMA,
            pltpu.SemaphoreType.DMA,
            pltpu.SemaphoreType.DMA,
        ],
    )
    def pass_a(src_hbm, p_hbm, q_hbm, out_hbm,
               idx0, idx1, rows0, rows1, qv0, qv1, rows_t, q_t, idx_t, st_v,
               semi0, semi1, semg0, semg1):
        wid = lax.axis_index("s") * NC + lax.axis_index("c")
        base = wid * ew
        idxs = [idx0, idx1]
        rows = [rows0, rows1]
        qvs = [qv0, qv1]
        semi = [semi0, semi1]
        semg = [semg0, semg1]

        def idx_copy(b, k):
            pltpu.async_copy(src_hbm.at[pl.ds(base + k * C, C)], idxs[b], semi[b])

        def idx_wait(b, k):
            pltpu.make_async_copy(
                src_hbm.at[pl.ds(base + k * C, C)], idxs[b], semi[b]).wait()

        def gath_start(b, k):
            pltpu.async_copy(p_hbm.at[idxs[b]], rows[b], semg[b])
            pltpu.async_copy(q_hbm.at[pl.ds(base + k * C, C)], qvs[b], semg[b])

        def gath_wait(b, k):
            pltpu.make_async_copy(p_hbm.at[idxs[b]], rows[b], semg[b]).wait()
            pltpu.make_async_copy(
                q_hbm.at[pl.ds(base + k * C, C)], qvs[b], semg[b]).wait()

        def rowloop(carry, nrows, rv, qv):
            def body(r, accs):
                out = list(accs)
                for j in range(nj):
                    sl = pl.ds(j * L, L)
                    m = rv[r, sl] + qv[r, sl]
                    out[j] = accs[j] + m
                    out[nj + j] = accs[nj + j] + m * m
                return tuple(out)

            return lax.fori_loop(0, nrows, body, carry)

        zero = jnp.zeros((L,), jnp.float32)
        carry = tuple(zero for _ in range(2 * nj))

        idx_copy(0, 0)
        idx_wait(0, 0)
        gath_start(0, 0)
        idx_copy(1, 1)

        def steady(k2, carry):
            for d in (0, 1):
                b = d
                ck = 2 * k2 + d
                idx_wait(1 - b, ck + 1)
                gath_start(1 - b, ck + 1)
                gath_wait(b, ck)
                idx_copy(b, ck + 2)
                carry = rowloop(carry, C, rows[b], qvs[b])
            return carry

        carry = lax.fori_loop(0, (nch - 2) // 2, steady, carry)

        idx_wait(1, nch - 1)
        gath_start(1, nch - 1)
        gath_wait(0, nch - 2)
        carry = rowloop(carry, C, rows[0], qvs[0])
        gath_wait(1, nch - 1)
        carry = rowloop(carry, C, rows[1], qvs[1])

        if tail:
            pltpu.sync_copy(src_hbm.at[pl.ds(base + nch * C, tail)], idx_t)
            pltpu.async_copy(p_hbm.at[idx_t], rows_t, semg0).wait()
            pltpu.sync_copy(q_hbm.at[pl.ds(base + nch * C, tail)], q_t)
            carry = rowloop(carry, tail, rows_t, q_t)

        for j in range(2 * nj):
            st_v[pl.ds(j * L, L)] = carry[j]
        pltpu.sync_copy(st_v, out_hbm.at[pl.ds(wid * 2 * h, 2 * h)])

    return pass_a


def _make_pass_b(e_edges, n_nodes, h):
    nj = h // L
    CB = 64
    ew = e_edges // NW
    nch = ew // CB
    tail = ew - nch * CB
    assert ew % 8 == 0 and tail % 8 == 0 and nch >= 6 and nch % 2 == 0
    tl = max(tail, 8)
    acc_rows = 10112
    tile_rows = acc_rows // NS
    zchunks = [(i * C, C) for i in range(tile_rows // C)]
    if tile_rows % C:
        zchunks.append(((tile_rows // C) * C, tile_rows % C))
    assert acc_rows >= n_nodes and tile_rows % 8 == 0

    @functools.partial(
        pl.kernel,
        out_type=jax.ShapeDtypeStruct((NC, acc_rows, h), jnp.float32),
        mesh=_mesh(),
        scratch_types=[
            pltpu.VMEM((CB,), jnp.int32),
            pltpu.VMEM((CB,), jnp.int32),
            pltpu.VMEM((CB,), jnp.int32),
            pltpu.VMEM((CB,), jnp.int32),
            pltpu.VMEM((CB, h), jnp.float32),
            pltpu.VMEM((CB, h), jnp.float32),
            pltpu.VMEM((CB, h), jnp.float32),
            pltpu.VMEM((CB, h), jnp.float32),
            pltpu.VMEM((tl, h), jnp.float32),
            pltpu.VMEM((tl, h), jnp.float32),
            pltpu.VMEM((tl,), jnp.int32),
            pltpu.VMEM((2, h), jnp.float32),
            pltpu.VMEM_SHARED((acc_rows, h), jnp.float32),
            pltpu.SemaphoreType.DMA,
            pltpu.SemaphoreType.DMA,
            pltpu.SemaphoreType.DMA,
            pltpu.SemaphoreType.DMA,
            pltpu.SemaphoreType.DMA,
            pltpu.SemaphoreType.DMA,
            pltpu.SemaphoreType.DMA,
            pltpu.SemaphoreType.DMA,
        ],
    )
    def pass_b(src_hbm, dst_hbm, t_hbm, q_hbm, st_hbm, out_hbm,
               idxs0, idxs1, idxd0, idxd1, rowst0, rowst1, qv0, qv1,
               rows_t, q_t, idx_t, st_v, acc,
               semi0, semi1, semd0, semd1, semg0, semg1, sems0, sems1):
        sc = lax.axis_index("c")
        tid = lax.axis_index("s")
        wid = tid * NC + sc
        base = wid * ew
        idxs = [idxs0, idxs1]
        idxd = [idxd0, idxd1]
        rowst = [rowst0, rowst1]
        qvs = [qv0, qv1]
        semi = [semi0, semi1]
        semd = [semd0, semd1]
        semg = [semg0, semg1]
        sems = [sems0, sems1]

        def zrow(r, _):
            for j in range(nj):
                rowst0[r, pl.ds(j * L, L)] = jnp.zeros((L,), jnp.float32)
                qv0[r, pl.ds(j * L, L)] = jnp.zeros((L,), jnp.float32)
            return 0

        lax.fori_loop(0, CB, zrow, 0)
        for off, sz in zchunks:
            r0 = tid * tile_rows + off
            if sz > CB:
                pltpu.sync_copy(rowst0, acc.at[pl.ds(r0, CB)])
                pltpu.sync_copy(qv0.at[pl.ds(0, sz - CB)],
                                acc.at[pl.ds(r0 + CB, sz - CB)])
            else:
                pltpu.sync_copy(rowst0.at[pl.ds(0, sz)], acc.at[pl.ds(r0, sz)])
        plsc.subcore_barrier()

        pltpu.sync_copy(st_hbm, st_v)
        s1 = [st_v[0, pl.ds(j * L, L)] for j in range(nj)]
        t1 = [st_v[1, pl.ds(j * L, L)] for j in range(nj)]

        def idx_copy(b, k):
            pltpu.async_copy(src_hbm.at[pl.ds(base + k * CB, CB)], idxs[b], semi[b])

        def idx_wait(b, k):
            pltpu.make_async_copy(
                src_hbm.at[pl.ds(base + k * CB, CB)], idxs[b], semi[b]).wait()

        def idxd_copy(b, k):
            pltpu.async_copy(dst_hbm.at[pl.ds(base + k * CB, CB)], idxd[b], semd[b])

        def idxd_wait(b, k):
            pltpu.make_async_copy(
                dst_hbm.at[pl.ds(base + k * CB, CB)], idxd[b], semd[b]).wait()

        def gath_start(b, k):
            pltpu.async_copy(t_hbm.at[idxs[b]], rowst[b], semg[b])
            pltpu.async_copy(q_hbm.at[pl.ds(base + k * CB, CB)], qvs[b], semg[b])

        def gath_wait(b, k):
            pltpu.make_async_copy(t_hbm.at[idxs[b]], rowst[b], semg[b]).wait()
            pltpu.make_async_copy(
                q_hbm.at[pl.ds(base + k * CB, CB)], qvs[b], semg[b]).wait()

        def scat_start(b):
            pltpu.async_copy(qvs[b], acc.at[idxd[b]], sems[b], add=True)

        def scat_wait(b):
            pltpu.make_async_copy(qvs[b], acc.at[pl.ds(0, CB)], sems[b]).wait()

        def rowloop(b):
            def body(r, _):
                for j in range(nj):
                    sl = pl.ds(j * L, L)
                    y = qvs[b][r, sl] * s1[j] + t1[j] + rowst[b][r, sl]
                    qvs[b][r, sl] = jnp.maximum(y, 0.0)
                return 0

            lax.fori_loop(0, CB, body, 0)

        def phase(b, ck, nxt1, nxt2, first):
            idx_wait(1 - b, nxt1)
            if not first:
                scat_wait(1 - b)
            idxd_copy(1 - b, nxt1)
            gath_start(1 - b, nxt1)
            gath_wait(b, ck)
            idx_copy(b, nxt2)
            rowloop(b)
            idxd_wait(b, ck)
            scat_start(b)

        idx_copy(0, 0)
        idx_wait(0, 0)
        idxd_copy(0, 0)
        gath_start(0, 0)
        idx_copy(1, 1)
        phase(0, 0, 1, 2, True)

        def steady(k2, _):
            c = 2 * k2 + 1
            phase(1, c, c + 1, c + 2, False)
            phase(0, c + 1, c + 2, c + 3, False)
            return 0

        lax.fori_loop(0, (nch - 4) // 2, steady, 0)

        phase(1, nch - 3, nch - 2, nch - 1, False)
        phase(0, nch - 2, nch - 1, nch - 1, False)
        phase(1, nch - 1, nch - 1, nch - 1, False)

        gath_wait(0, nch - 1)
        idx_wait(1, nch - 1)
        idxd_wait(0, nch - 1)
        scat_wait(1)

        if tail:
            pltpu.sync_copy(src_hbm.at[pl.ds(base + nch * CB, tail)], idx_t)
            pltpu.async_copy(t_hbm.at[idx_t], rows_t, semg0).wait()
            pltpu.sync_copy(q_hbm.at[pl.ds(base + nch * CB, tail)], q_t)

            def tbody(r, _):
                for j in range(nj):
                    sl = pl.ds(j * L, L)
                    y = q_t[r, sl] * s1[j] + t1[j] + rows_t[r, sl]
                    q_t[r, sl] = jnp.maximum(y, 0.0)
                return 0

            lax.fori_loop(0, tail, tbody, 0)
            pltpu.sync_copy(dst_hbm.at[pl.ds(base + nch * CB, tail)], idx_t)
            pltpu.sync_copy(q_t, acc.at[idx_t], add=True)

        plsc.subcore_barrier()

        for off, sz in zchunks:
            r0 = tid * tile_rows + off
            if sz > CB:
                pltpu.sync_copy(acc.at[pl.ds(r0, CB)], rowst0)
                pltpu.sync_copy(rowst0, out_hbm.at[sc, pl.ds(r0, CB)])
                pltpu.sync_copy(acc.at[pl.ds(r0 + CB, sz - CB)],
                                qv0.at[pl.ds(0, sz - CB)])
                pltpu.sync_copy(qv0.at[pl.ds(0, sz - CB)],
                                out_hbm.at[sc, pl.ds(r0 + CB, sz - CB)])
            else:
                pltpu.sync_copy(acc.at[pl.ds(r0, sz)], rowst0.at[pl.ds(0, sz)])
                pltpu.sync_copy(rowst0.at[pl.ds(0, sz)],
                                out_hbm.at[sc, pl.ds(r0, sz)])

    return pass_b


def kernel(features, x_atom, edge_attr, edge_index, W1, b1, g1, be1, W2, b2, g2, be2):
    n, h = features.shape
    a = x_atom.shape[1]
    e, bf = edge_attr.shape

    src = edge_index[0]
    dst = edge_index[1]
    w1h = W1[:h]
    w1x = W1[h:h + a]
    w1w = W1[h + a:]

    bn = 1000
    p = pl.pallas_call(
        _a1_body,
        grid=(n // bn,),
        in_specs=[
            pl.BlockSpec((bn, h), lambda i: (i, 0)),
            pl.BlockSpec((bn, a), lambda i: (i, 0)),
            pl.BlockSpec((h, h), lambda i: (0, 0)),
            pl.BlockSpec((a, h), lambda i: (0, 0)),
        ],
        out_specs=pl.BlockSpec((bn, h), lambda i: (i, 0)),
        out_shape=jax.ShapeDtypeStruct((n, h), jnp.float32),
    )(features, x_atom, w1h, w1x)

    be_ = 4000
    q = pl.pallas_call(
        _a2_body,
        grid=(e // be_,),
        in_specs=[
            pl.BlockSpec((be_, bf), lambda i: (i, 0)),
            pl.BlockSpec((bf, h), lambda i: (0, 0)),
            pl.BlockSpec((1, h), lambda i: (0, 0)),
        ],
        out_specs=pl.BlockSpec((be_, h), lambda i: (i, 0)),
        out_shape=jax.ShapeDtypeStruct((e, h), jnp.float32),
    )(edge_attr, w1w, b1.reshape(1, h))

    parts = _make_pass_a(e, h)(src, p, q).reshape(NW, 2 * h)
    sums = jnp.sum(parts[:, :h], axis=0)
    sqs = jnp.sum(parts[:, h:], axis=0)
    mu1 = sums / e
    var1 = sqs / e - mu1 * mu1
    s1 = g1 / jnp.sqrt(var1 + EPS)
    t1 = be1 - mu1 * s1
    st1 = jnp.stack([s1, t1])

    t_tab = pl.pallas_call(
        _t_body,
        grid=(n // bn,),
        in_specs=[
            pl.BlockSpec((bn, h), lambda i: (i, 0)),
            pl.BlockSpec((bn, h), lambda i: (i, 0)),
            pl.BlockSpec((2, h), lambda i: (0, 0)),
        ],
        out_specs=pl.BlockSpec((bn, h), lambda i: (i, 0)),
        out_shape=jax.ShapeDtypeStruct((n, h), jnp.float32),
    )(p, features, st1)

    acc_rows = 10112
    hsc = _make_pass_b(e, n, h)(src, dst, t_tab, q, st1)
    h2 = hsc.reshape(NC * acc_rows, h)

    bn_e = 632
    ng_e = acc_rows // bn_e
    y, st_parts = pl.pallas_call(
        functools.partial(_e1_body, n, bn_e),
        grid=(ng_e,),
        in_specs=[
            pl.BlockSpec((bn_e, h), lambda i: (i, 0)),
            pl.BlockSpec((bn_e, h), lambda i: (i + 16, 0)),
            pl.BlockSpec((h, h), lambda i: (0, 0)),
            pl.BlockSpec((1, h), lambda i: (0, 0)),
        ],
        out_specs=[
            pl.BlockSpec((bn_e, h), lambda i: (i, 0)),
            pl.BlockSpec((1, 2, h), lambda i: (i, 0, 0)),
        ],
        out_shape=[
            jax.ShapeDtypeStruct((acc_rows, h), jnp.float32),
            jax.ShapeDtypeStruct((ng_e, 2, h), jnp.float32),
        ],
    )(h2, h2, W2, b2.reshape(1, h))

    mu2 = jnp.sum(st_parts[:, 0, :], axis=0) / n
    var2 = jnp.sum(st_parts[:, 1, :], axis=0) / n - mu2 * mu2
    s2 = g2 / jnp.sqrt(var2 + EPS)
    t2 = be2 - mu2 * s2
    st2 = jnp.stack([s2, t2])

    out = pl.pallas_call(
        _e2_body,
        grid=(n // bn,),
        in_specs=[
            pl.BlockSpec((bn, h), lambda i: (i, 0)),
            pl.BlockSpec((bn, h), lambda i: (i, 0)),
            pl.BlockSpec((2, h), lambda i: (0, 0)),
        ],
        out_specs=pl.BlockSpec((bn, h), lambda i: (i, 0)),
        out_shape=jax.ShapeDtypeStruct((n, h), jnp.float32),
    )(y, features, st2)
    return out


# speedup vs baseline: 5.2073x; 1.0073x over previous
import functools

import jax
import jax.numpy as jnp
from jax import lax
from jax.experimental import pallas as pl
from jax.experimental.pallas import tpu as pltpu
from jax.experimental.pallas import tpu_sc as plsc

NC = 2
NS = 16
NW = NC * NS
L = 16
C = 128
EPS = 1e-5


def _mesh():
    return plsc.VectorSubcoreMesh(
        core_axis_name="c", subcore_axis_name="s", num_cores=NC, num_subcores=NS
    )


def _a1_body(f_ref, x_ref, wh_ref, wx_ref, o_ref):
    o_ref[...] = (
        jnp.dot(f_ref[...], wh_ref[...], preferred_element_type=jnp.float32)
        + jnp.dot(x_ref[...], wx_ref[...], preferred_element_type=jnp.float32)
    )


def _a2_body(e_ref, w_ref, b_ref, o_ref):
    o_ref[...] = (
        jnp.dot(e_ref[...], w_ref[...], preferred_element_type=jnp.float32)
        + b_ref[...]
    )


def _t_body(e_edges, parts_ref, g1_ref, be1_ref, p_ref, f_ref, o_ref, st_ref):
    parts = parts_ref[...]
    h = g1_ref.shape[1]
    mu = jnp.sum(parts[:, :h], axis=0, keepdims=True) / e_edges
    var = jnp.sum(parts[:, h:], axis=0, keepdims=True) / e_edges - mu * mu
    s1 = g1_ref[...] / jnp.sqrt(var + EPS)
    t1 = be1_ref[...] - mu * s1
    o_ref[...] = p_ref[...] * s1 + f_ref[...]
    st_ref[0:1, :] = s1
    st_ref[1:2, :] = t1


def _e_body(nvalid, bs1, np1, bs2, c0_ref, c1_ref, w2_ref, b2_ref, g2_ref,
            be2_ref, f_ref, o_ref, y_s, stat_s, st_s):
    i = pl.program_id(0)

    @pl.when(i == 0)
    def _():
        stat_s[...] = jnp.zeros_like(stat_s)

    @pl.when(i < np1)
    def _():
        y = (
            jnp.dot(c0_ref[...] + c1_ref[...], w2_ref[...],
                    preferred_element_type=jnp.float32)
            + b2_ref[...]
        )
        y_s[pl.ds(i * bs1, bs1), :] = y
        rows = lax.broadcasted_iota(jnp.int32, (bs1, 1), 0) + i * bs1
        ym = jnp.where(rows < nvalid, y, 0.0)
        stat_s[0:1, :] += jnp.sum(ym, axis=0, keepdims=True)
        stat_s[1:2, :] += jnp.sum(ym * ym, axis=0, keepdims=True)

    @pl.when(i == np1)
    def _():
        mu = stat_s[0:1, :] / nvalid
        var = stat_s[1:2, :] / nvalid - mu * mu
        s2 = g2_ref[...] / jnp.sqrt(var + EPS)
        st_s[0:1, :] = s2
        st_s[1:2, :] = be2_ref[...] - mu * s2

    @pl.when(i >= np1)
    def _():
        j = i - np1
        y = y_s[pl.ds(j * bs2, bs2), :]
        o_ref[...] = jnp.maximum(
            y * st_s[0:1, :] + st_s[1:2, :] + f_ref[...], 0.0)


def _make_pass_a(e_edges, h):
    nj = h // L
    ew = e_edges // NW
    nch = ew // C
    tail = ew - nch * C
    assert ew % 8 == 0 and tail % 8 == 0 and nch >= 4 and nch % 2 == 0
    tl = max(tail, 8)

    @functools.partial(
        pl.kernel,
        out_type=jax.ShapeDtypeStruct((NW * 2 * h,), jnp.float32),
        mesh=_mesh(),
        scratch_types=[
            pltpu.VMEM((C,), jnp.int32),
            pltpu.VMEM((C,), jnp.int32),
            pltpu.VMEM((C, h), jnp.float32),
            pltpu.VMEM((C, h), jnp.float32),
            pltpu.VMEM((C, h), jnp.float32),
            pltpu.VMEM((C, h), jnp.float32),
            pltpu.VMEM((tl, h), jnp.float32),
            pltpu.VMEM((tl, h), jnp.float32),
            pltpu.VMEM((tl,), jnp.int32),
            pltpu.VMEM((2 * h,), jnp.float32),
            pltpu.SemaphoreType.DMA,
            pltpu.SemaphoreType.DMA,
            pltpu.SemaphoreType.DMA,
            pltpu.SemaphoreType.DMA,
        ],
    )
    def pass_a(src_hbm, p_hbm, q_hbm, out_hbm,
               idx0, idx1, rows0, rows1, qv0, qv1, rows_t, q_t, idx_t, st_v,
               semi0, semi1, semg0, semg1):
        wid = lax.axis_index("s") * NC + lax.axis_index("c")
        base = wid * ew
        idxs = [idx0, idx1]
        rows = [rows0, rows1]
        qvs = [qv0, qv1]
        semi = [semi0, semi1]
        semg = [semg0, semg1]

        def idx_copy(b, k):
            pltpu.async_copy(src_hbm.at[pl.ds(base + k * C, C)], idxs[b], semi[b])

        def idx_wait(b, k):
            pltpu.make_async_copy(
                src_hbm.at[pl.ds(base + k * C, C)], idxs[b], semi[b]).wait()

        def gath_start(b, k):
            pltpu.async_copy(p_hbm.at[idxs[b]], rows[b], semg[b])
            pltpu.async_copy(q_hbm.at[pl.ds(base + k * C, C)], qvs[b], semg[b])

        def gath_wait(b, k):
            pltpu.make_async_copy(p_hbm.at[idxs[b]], rows[b], semg[b]).wait()
            pltpu.make_async_copy(
                q_hbm.at[pl.ds(base + k * C, C)], qvs[b], semg[b]).wait()

        def rowloop(carry, nrows, rv, qv):
            def body(r, accs):
                out = list(accs)
                for j in range(nj):
                    sl = pl.ds(j * L, L)
                    m = rv[r, sl] + qv[r, sl]
                    out[j] = accs[j] + m
                    out[nj + j] = accs[nj + j] + m * m
                return tuple(out)

            return lax.fori_loop(0, nrows, body, carry)

        zero = jnp.zeros((L,), jnp.float32)
        carry = tuple(zero for _ in range(2 * nj))

        idx_copy(0, 0)
        idx_wait(0, 0)
        gath_start(0, 0)
        idx_copy(1, 1)

        def steady(k2, carry):
            for d in (0, 1):
                b = d
                ck = 2 * k2 + d
                idx_wait(1 - b, ck + 1)
                gath_start(1 - b, ck + 1)
                gath_wait(b, ck)
                idx_copy(b, ck + 2)
                carry = rowloop(carry, C, rows[b], qvs[b])
            return carry

        carry = lax.fori_loop(0, (nch - 2) // 2, steady, carry)

        idx_wait(1, nch - 1)
        gath_start(1, nch - 1)
        gath_wait(0, nch - 2)
        carry = rowloop(carry, C, rows[0], qvs[0])
        gath_wait(1, nch - 1)
        carry = rowloop(carry, C, rows[1], qvs[1])

        if tail:
            pltpu.sync_copy(src_hbm.at[pl.ds(base + nch * C, tail)], idx_t)
            pltpu.async_copy(p_hbm.at[idx_t], rows_t, semg0).wait()
            pltpu.sync_copy(q_hbm.at[pl.ds(base + nch * C, tail)], q_t)
            carry = rowloop(carry, tail, rows_t, q_t)

        for j in range(2 * nj):
            st_v[pl.ds(j * L, L)] = carry[j]
        pltpu.sync_copy(st_v, out_hbm.at[pl.ds(wid * 2 * h, 2 * h)])

    return pass_a


def _make_pass_b(e_edges, n_nodes, h):
    nj = h // L
    CB = 64
    ew = e_edges // NW
    nch = ew // CB
    tail = ew - nch * CB
    assert ew % 8 == 0 and tail % 8 == 0 and nch >= 6 and nch % 2 == 0
    tl = max(tail, 8)
    acc_rows = 10112
    tile_rows = acc_rows // NS
    zchunks = [(i * C, C) for i in range(tile_rows // C)]
    if tile_rows % C:
        zchunks.append(((tile_rows // C) * C, tile_rows % C))
    assert acc_rows >= n_nodes and tile_rows % 8 == 0

    @functools.partial(
        pl.kernel,
        out_type=jax.ShapeDtypeStruct((NC, acc_rows, h), jnp.float32),
        mesh=_mesh(),
        scratch_types=[
            pltpu.VMEM((CB,), jnp.int32),
            pltpu.VMEM((CB,), jnp.int32),
            pltpu.VMEM((CB,), jnp.int32),
            pltpu.VMEM((CB,), jnp.int32),
            pltpu.VMEM((CB, h), jnp.float32),
            pltpu.VMEM((CB, h), jnp.float32),
            pltpu.VMEM((CB, h), jnp.float32),
            pltpu.VMEM((CB, h), jnp.float32),
            pltpu.VMEM((tl, h), jnp.float32),
            pltpu.VMEM((tl, h), jnp.float32),
            pltpu.VMEM((tl,), jnp.int32),
            pltpu.VMEM((2, h), jnp.float32),
            pltpu.VMEM_SHARED((acc_rows, h), jnp.float32),
            pltpu.SemaphoreType.DMA,
            pltpu.SemaphoreType.DMA,
            pltpu.SemaphoreType.DMA,
            pltpu.SemaphoreType.DMA,
            pltpu.SemaphoreType.DMA,
            pltpu.SemaphoreType.DMA,
            pltpu.SemaphoreType.DMA,
            pltpu.SemaphoreType.DMA,
        ],
    )
    def pass_b(src_hbm, dst_hbm, t_hbm, q_hbm, st_hbm, out_hbm,
               idxs0, idxs1, idxd0, idxd1, rowst0, rowst1, qv0, qv1,
               rows_t, q_t, idx_t, st_v, acc,
               semi0, semi1, semd0, semd1, semg0, semg1, sems0, sems1):
        sc = lax.axis_index("c")
        tid = lax.axis_index("s")
        wid = tid * NC + sc
        base = wid * ew
        idxs = [idxs0, idxs1]
        idxd = [idxd0, idxd1]
        rowst = [rowst0, rowst1]
        qvs = [qv0, qv1]
        semi = [semi0, semi1]
        semd = [semd0, semd1]
        semg = [semg0, semg1]
        sems = [sems0, sems1]

        def zrow(r, _):
            for j in range(nj):
                rowst0[r, pl.ds(j * L, L)] = jnp.zeros((L,), jnp.float32)
                qv0[r, pl.ds(j * L, L)] = jnp.zeros((L,), jnp.float32)
            return 0

        lax.fori_loop(0, CB, zrow, 0)
        for off, sz in zchunks:
            r0 = tid * tile_rows + off
            if sz > CB:
                pltpu.sync_copy(rowst0, acc.at[pl.ds(r0, CB)])
                pltpu.sync_copy(qv0.at[pl.ds(0, sz - CB)],
                                acc.at[pl.ds(r0 + CB, sz - CB)])
            else:
                pltpu.sync_copy(rowst0.at[pl.ds(0, sz)], acc.at[pl.ds(r0, sz)])
        plsc.subcore_barrier()

        pltpu.sync_copy(st_hbm, st_v)
        s1 = [st_v[0, pl.ds(j * L, L)] for j in range(nj)]
        t1 = [st_v[1, pl.ds(j * L, L)] for j in range(nj)]

        def idx_copy(b, k):
            pltpu.async_copy(src_hbm.at[pl.ds(base + k * CB, CB)], idxs[b], semi[b])

        def idx_wait(b, k):
            pltpu.make_async_copy(
                src_hbm.at[pl.ds(base + k * CB, CB)], idxs[b], semi[b]).wait()

        def idxd_copy(b, k):
            pltpu.async_copy(dst_hbm.at[pl.ds(base + k * CB, CB)], idxd[b], semd[b])

        def idxd_wait(b, k):
            pltpu.make_async_copy(
                dst_hbm.at[pl.ds(base + k * CB, CB)], idxd[b], semd[b]).wait()

        def gath_start(b, k):
            pltpu.async_copy(t_hbm.at[idxs[b]], rowst[b], semg[b])
            pltpu.async_copy(q_hbm.at[pl.ds(base + k * CB, CB)], qvs[b], semg[b])

        def gath_wait(b, k):
            pltpu.make_async_copy(t_hbm.at[idxs[b]], rowst[b], semg[b]).wait()
            pltpu.make_async_copy(
                q_hbm.at[pl.ds(base + k * CB, CB)], qvs[b], semg[b]).wait()

        def scat_start(b):
            pltpu.async_copy(qvs[b], acc.at[idxd[b]], sems[b], add=True)

        def scat_wait(b):
            pltpu.make_async_copy(qvs[b], acc.at[pl.ds(0, CB)], sems[b]).wait()

        def rowloop(b):
            def body(r, _):
                for j in range(nj):
                    sl = pl.ds(j * L, L)
                    y = qvs[b][r, sl] * s1[j] + t1[j] + rowst[b][r, sl]
                    qvs[b][r, sl] = jnp.maximum(y, 0.0)
                return 0

            lax.fori_loop(0, CB, body, 0)

        def phase(b, ck, nxt1, nxt2, first):
            idx_wait(1 - b, nxt1)
            if not first:
                scat_wait(1 - b)
            idxd_copy(1 - b, nxt1)
            gath_start(1 - b, nxt1)
            gath_wait(b, ck)
            idx_copy(b, nxt2)
            rowloop(b)
            idxd_wait(b, ck)
            scat_start(b)

        idx_copy(0, 0)
        idx_wait(0, 0)
        idxd_copy(0, 0)
        gath_start(0, 0)
        idx_copy(1, 1)
        phase(0, 0, 1, 2, True)

        def steady(k2, _):
            c = 2 * k2 + 1
            phase(1, c, c + 1, c + 2, False)
            phase(0, c + 1, c + 2, c + 3, False)
            return 0

        lax.fori_loop(0, (nch - 4) // 2, steady, 0)

        phase(1, nch - 3, nch - 2, nch - 1, False)
        phase(0, nch - 2, nch - 1, nch - 1, False)
        phase(1, nch - 1, nch - 1, nch - 1, False)

        gath_wait(0, nch - 1)
        idx_wait(1, nch - 1)
        idxd_wait(0, nch - 1)
        scat_wait(1)

        if tail:
            pltpu.sync_copy(src_hbm.at[pl.ds(base + nch * CB, tail)], idx_t)
            pltpu.async_copy(t_hbm.at[idx_t], rows_t, semg0).wait()
            pltpu.sync_copy(q_hbm.at[pl.ds(base + nch * CB, tail)], q_t)

            def tbody(r, _):
                for j in range(nj):
                    sl = pl.ds(j * L, L)
                    y = q_t[r, sl] * s1[j] + t1[j] + rows_t[r, sl]
                    q_t[r, sl] = jnp.maximum(y, 0.0)
                return 0

            lax.fori_loop(0, tail, tbody, 0)
            pltpu.sync_copy(dst_hbm.at[pl.ds(base + nch * CB, tail)], idx_t)
            pltpu.sync_copy(q_t, acc.at[idx_t], add=True)

        plsc.subcore_barrier()

        for off, sz in zchunks:
            r0 = tid * tile_rows + off
            if sz > CB:
                pltpu.sync_copy(acc.at[pl.ds(r0, CB)], rowst0)
                pltpu.sync_copy(rowst0, out_hbm.at[sc, pl.ds(r0, CB)])
                pltpu.sync_copy(acc.at[pl.ds(r0 + CB, sz - CB)],
                                qv0.at[pl.ds(0, sz - CB)])
                pltpu.sync_copy(qv0.at[pl.ds(0, sz - CB)],
                                out_hbm.at[sc, pl.ds(r0 + CB, sz - CB)])
            else:
                pltpu.sync_copy(acc.at[pl.ds(r0, sz)], rowst0.at[pl.ds(0, sz)])
                pltpu.sync_copy(rowst0.at[pl.ds(0, sz)],
                                out_hbm.at[sc, pl.ds(r0, sz)])

    return pass_b


def kernel(features, x_atom, edge_attr, edge_index, W1, b1, g1, be1, W2, b2, g2, be2):
    n, h = features.shape
    a = x_atom.shape[1]
    e, bf = edge_attr.shape

    src = edge_index[0]
    dst = edge_index[1]
    w1h = W1[:h]
    w1x = W1[h:h + a]
    w1w = W1[h + a:]

    bn = 1000
    p = pl.pallas_call(
        _a1_body,
        grid=(n // bn,),
        in_specs=[
            pl.BlockSpec((bn, h), lambda i: (i, 0)),
            pl.BlockSpec((bn, a), lambda i: (i, 0)),
            pl.BlockSpec((h, h), lambda i: (0, 0)),
            pl.BlockSpec((a, h), lambda i: (0, 0)),
        ],
        out_specs=pl.BlockSpec((bn, h), lambda i: (i, 0)),
        out_shape=jax.ShapeDtypeStruct((n, h), jnp.float32),
    )(features, x_atom, w1h, w1x)

    be_ = 4000
    q = pl.pallas_call(
        _a2_body,
        grid=(e // be_,),
        in_specs=[
            pl.BlockSpec((be_, bf), lambda i: (i, 0)),
            pl.BlockSpec((bf, h), lambda i: (0, 0)),
            pl.BlockSpec((1, h), lambda i: (0, 0)),
        ],
        out_specs=pl.BlockSpec((be_, h), lambda i: (i, 0)),
        out_shape=jax.ShapeDtypeStruct((e, h), jnp.float32),
    )(edge_attr, w1w, b1.reshape(1, h))

    parts = _make_pass_a(e, h)(src, p, q).reshape(NW, 2 * h)

    t_tab, st1 = pl.pallas_call(
        functools.partial(_t_body, e),
        grid=(n // bn,),
        in_specs=[
            pl.BlockSpec((NW, 2 * h), lambda i: (0, 0)),
            pl.BlockSpec((1, h), lambda i: (0, 0)),
            pl.BlockSpec((1, h), lambda i: (0, 0)),
            pl.BlockSpec((bn, h), lambda i: (i, 0)),
            pl.BlockSpec((bn, h), lambda i: (i, 0)),
        ],
        out_specs=[
            pl.BlockSpec((bn, h), lambda i: (i, 0)),
            pl.BlockSpec((2, h), lambda i: (0, 0)),
        ],
        out_shape=[
            jax.ShapeDtypeStruct((n, h), jnp.float32),
            jax.ShapeDtypeStruct((2, h), jnp.float32),
        ],
    )(parts, g1.reshape(1, h), be1.reshape(1, h), p, features)

    acc_rows = 10112
    hsc = _make_pass_b(e, n, h)(src, dst, t_tab, q, st1)
    h2 = hsc.reshape(NC * acc_rows, h)

    bs1 = 632
    np1 = acc_rows // bs1
    bs2 = 1000
    np2 = n // bs2
    out = pl.pallas_call(
        functools.partial(_e_body, n, bs1, np1, bs2),
        grid=(np1 + np2,),
        in_specs=[
            pl.BlockSpec((bs1, h), lambda i: (jnp.minimum(i, np1 - 1), 0)),
            pl.BlockSpec((bs1, h), lambda i: (jnp.minimum(i, np1 - 1) + np1, 0)),
            pl.BlockSpec((h, h), lambda i: (0, 0)),
            pl.BlockSpec((1, h), lambda i: (0, 0)),
            pl.BlockSpec((1, h), lambda i: (0, 0)),
            pl.BlockSpec((1, h), lambda i: (0, 0)),
            pl.BlockSpec((bs2, h), lambda i: (jnp.maximum(i - np1, 0), 0)),
        ],
        out_specs=pl.BlockSpec((bs2, h), lambda i: (jnp.maximum(i - np1, 0), 0)),
        out_shape=jax.ShapeDtypeStruct((n, h), jnp.float32),
        scratch_shapes=[
            pltpu.VMEM((acc_rows, h), jnp.float32),
            pltpu.VMEM((2, h), jnp.float32),
            pltpu.VMEM((2, h), jnp.float32),
        ],
    )(h2, h2, W2, b2.reshape(1, h), g2.reshape(1, h), be2.reshape(1, h),
      features)
    return out


# speedup vs baseline: 6.5181x; 1.2517x over previous
import functools

import jax
import jax.numpy as jnp
from jax import lax
from jax.experimental import pallas as pl
from jax.experimental.pallas import tpu as pltpu
from jax.experimental.pallas import tpu_sc as plsc

NC = 2
NS = 16
NW = NC * NS
L = 16
C = 128
EPS = 1e-5


def _mesh():
    return plsc.VectorSubcoreMesh(
        core_axis_name="c", subcore_axis_name="s", num_cores=NC, num_subcores=NS
    )


def _a1_body(f_ref, x_ref, wh_ref, wx_ref, o_ref):
    o_ref[...] = (
        jnp.dot(f_ref[...], wh_ref[...], preferred_element_type=jnp.float32)
        + jnp.dot(x_ref[...], wx_ref[...], preferred_element_type=jnp.float32)
    )


def _a2_body(et_ref, w_ref, b_ref, o_ref):
    o_ref[...] = (
        lax.dot_general(et_ref[...], w_ref[...], (((0,), (0,)), ((), ())),
                        preferred_element_type=jnp.float32)
        + b_ref[...]
    )


def _t_body(e_edges, parts_ref, g1_ref, be1_ref, p_ref, f_ref, o_ref, st_ref):
    parts = parts_ref[...]
    h = g1_ref.shape[1]
    mu = jnp.sum(parts[:, :h], axis=0, keepdims=True) / e_edges
    var = jnp.sum(parts[:, h:], axis=0, keepdims=True) / e_edges - mu * mu
    s1 = g1_ref[...] / jnp.sqrt(var + EPS)
    t1 = be1_ref[...] - mu * s1
    o_ref[...] = p_ref[...] * s1 + f_ref[...]
    st_ref[0:1, :] = s1
    st_ref[1:2, :] = t1


def _e_body(nvalid, bs1, np1, bs2, c0_ref, c1_ref, w2_ref, b2_ref, g2_ref,
            be2_ref, f_ref, o_ref, y_s, stat_s, st_s):
    i = pl.program_id(0)

    @pl.when(i == 0)
    def _():
        stat_s[...] = jnp.zeros_like(stat_s)

    @pl.when(i < np1)
    def _():
        y = (
            jnp.dot(c0_ref[...] + c1_ref[...], w2_ref[...],
                    preferred_element_type=jnp.float32)
            + b2_ref[...]
        )
        y_s[pl.ds(i * bs1, bs1), :] = y
        rows = lax.broadcasted_iota(jnp.int32, (bs1, 1), 0) + i * bs1
        ym = jnp.where(rows < nvalid, y, 0.0)
        stat_s[0:1, :] += jnp.sum(ym, axis=0, keepdims=True)
        stat_s[1:2, :] += jnp.sum(ym * ym, axis=0, keepdims=True)

    @pl.when(i == np1)
    def _():
        mu = stat_s[0:1, :] / nvalid
        var = stat_s[1:2, :] / nvalid - mu * mu
        s2 = g2_ref[...] / jnp.sqrt(var + EPS)
        st_s[0:1, :] = s2
        st_s[1:2, :] = be2_ref[...] - mu * s2

    @pl.when(i >= np1)
    def _():
        j = i - np1
        y = y_s[pl.ds(j * bs2, bs2), :]
        o_ref[...] = jnp.maximum(
            y * st_s[0:1, :] + st_s[1:2, :] + f_ref[...], 0.0)


def _make_pass_a(e_edges, h):
    nj = h // L
    ew = e_edges // NW
    nch = ew // C
    tail = ew - nch * C
    assert ew % 8 == 0 and tail % 8 == 0 and nch >= 4 and nch % 2 == 0
    tl = max(tail, 8)

    @functools.partial(
        pl.kernel,
        out_type=jax.ShapeDtypeStruct((NW * 2 * h,), jnp.float32),
        mesh=_mesh(),
        scratch_types=[
            pltpu.VMEM((C,), jnp.int32),
            pltpu.VMEM((C,), jnp.int32),
            pltpu.VMEM((C, h), jnp.float32),
            pltpu.VMEM((C, h), jnp.float32),
            pltpu.VMEM((C, h), jnp.float32),
            pltpu.VMEM((C, h), jnp.float32),
            pltpu.VMEM((tl, h), jnp.float32),
            pltpu.VMEM((tl, h), jnp.float32),
            pltpu.VMEM((tl,), jnp.int32),
            pltpu.VMEM((2 * h,), jnp.float32),
            pltpu.SemaphoreType.DMA,
            pltpu.SemaphoreType.DMA,
            pltpu.SemaphoreType.DMA,
            pltpu.SemaphoreType.DMA,
        ],
    )
    def pass_a(src_hbm, p_hbm, q_hbm, out_hbm,
               idx0, idx1, rows0, rows1, qv0, qv1, rows_t, q_t, idx_t, st_v,
               semi0, semi1, semg0, semg1):
        wid = lax.axis_index("s") * NC + lax.axis_index("c")
        base = wid * ew
        idxs = [idx0, idx1]
        rows = [rows0, rows1]
        qvs = [qv0, qv1]
        semi = [semi0, semi1]
        semg = [semg0, semg1]

        def idx_copy(b, k):
            pltpu.async_copy(src_hbm.at[pl.ds(base + k * C, C)], idxs[b], semi[b])

        def idx_wait(b, k):
            pltpu.make_async_copy(
                src_hbm.at[pl.ds(base + k * C, C)], idxs[b], semi[b]).wait()

        def gath_start(b, k):
            pltpu.async_copy(p_hbm.at[idxs[b]], rows[b], semg[b])
            pltpu.async_copy(q_hbm.at[pl.ds(base + k * C, C)], qvs[b], semg[b])

        def gath_wait(b, k):
            pltpu.make_async_copy(p_hbm.at[idxs[b]], rows[b], semg[b]).wait()
            pltpu.make_async_copy(
                q_hbm.at[pl.ds(base + k * C, C)], qvs[b], semg[b]).wait()

        def rowloop(carry, nrows, rv, qv):
            def body(r, accs):
                out = list(accs)
                for j in range(nj):
                    sl = pl.ds(j * L, L)
                    m = rv[r, sl] + qv[r, sl]
                    out[j] = accs[j] + m
                    out[nj + j] = accs[nj + j] + m * m
                return tuple(out)

            return lax.fori_loop(0, nrows, body, carry)

        zero = jnp.zeros((L,), jnp.float32)
        carry = tuple(zero for _ in range(2 * nj))

        idx_copy(0, 0)
        idx_wait(0, 0)
        gath_start(0, 0)
        idx_copy(1, 1)

        def steady(k2, carry):
            for d in (0, 1):
                b = d
                ck = 2 * k2 + d
                idx_wait(1 - b, ck + 1)
                gath_start(1 - b, ck + 1)
                gath_wait(b, ck)
                idx_copy(b, ck + 2)
                carry = rowloop(carry, C, rows[b], qvs[b])
            return carry

        carry = lax.fori_loop(0, (nch - 2) // 2, steady, carry)

        idx_wait(1, nch - 1)
        gath_start(1, nch - 1)
        gath_wait(0, nch - 2)
        carry = rowloop(carry, C, rows[0], qvs[0])
        gath_wait(1, nch - 1)
        carry = rowloop(carry, C, rows[1], qvs[1])

        if tail:
            pltpu.sync_copy(src_hbm.at[pl.ds(base + nch * C, tail)], idx_t)
            pltpu.async_copy(p_hbm.at[idx_t], rows_t, semg0).wait()
            pltpu.sync_copy(q_hbm.at[pl.ds(base + nch * C, tail)], q_t)
            carry = rowloop(carry, tail, rows_t, q_t)

        for j in range(2 * nj):
            st_v[pl.ds(j * L, L)] = carry[j]
        pltpu.sync_copy(st_v, out_hbm.at[pl.ds(wid * 2 * h, 2 * h)])

    return pass_a


def _make_pass_b(e_edges, n_nodes, h):
    nj = h // L
    CB = 64
    ew = e_edges // NW
    nch = ew // CB
    tail = ew - nch * CB
    assert ew % 8 == 0 and tail % 8 == 0 and nch >= 6 and nch % 2 == 0
    tl = max(tail, 8)
    acc_rows = 10112
    tile_rows = acc_rows // NS
    zchunks = [(i * C, C) for i in range(tile_rows // C)]
    if tile_rows % C:
        zchunks.append(((tile_rows // C) * C, tile_rows % C))
    assert acc_rows >= n_nodes and tile_rows % 8 == 0

    @functools.partial(
        pl.kernel,
        out_type=jax.ShapeDtypeStruct((NC, acc_rows, h), jnp.float32),
        mesh=_mesh(),
        scratch_types=[
            pltpu.VMEM((CB,), jnp.int32),
            pltpu.VMEM((CB,), jnp.int32),
            pltpu.VMEM((CB,), jnp.int32),
            pltpu.VMEM((CB,), jnp.int32),
            pltpu.VMEM((CB, h), jnp.float32),
            pltpu.VMEM((CB, h), jnp.float32),
            pltpu.VMEM((CB, h), jnp.float32),
            pltpu.VMEM((CB, h), jnp.float32),
            pltpu.VMEM((tl, h), jnp.float32),
            pltpu.VMEM((tl, h), jnp.float32),
            pltpu.VMEM((tl,), jnp.int32),
            pltpu.VMEM((2, h), jnp.float32),
            pltpu.VMEM_SHARED((acc_rows, h), jnp.float32),
            pltpu.SemaphoreType.DMA,
            pltpu.SemaphoreType.DMA,
            pltpu.SemaphoreType.DMA,
            pltpu.SemaphoreType.DMA,
            pltpu.SemaphoreType.DMA,
            pltpu.SemaphoreType.DMA,
            pltpu.SemaphoreType.DMA,
            pltpu.SemaphoreType.DMA,
        ],
    )
    def pass_b(src_hbm, dst_hbm, t_hbm, q_hbm, st_hbm, out_hbm,
               idxs0, idxs1, idxd0, idxd1, rowst0, rowst1, qv0, qv1,
               rows_t, q_t, idx_t, st_v, acc,
               semi0, semi1, semd0, semd1, semg0, semg1, sems0, sems1):
        sc = lax.axis_index("c")
        tid = lax.axis_index("s")
        wid = tid * NC + sc
        base = wid * ew
        idxs = [idxs0, idxs1]
        idxd = [idxd0, idxd1]
        rowst = [rowst0, rowst1]
        qvs = [qv0, qv1]
        semi = [semi0, semi1]
        semd = [semd0, semd1]
        semg = [semg0, semg1]
        sems = [sems0, sems1]

        def zrow(r, _):
            for j in range(nj):
                rowst0[r, pl.ds(j * L, L)] = jnp.zeros((L,), jnp.float32)
                qv0[r, pl.ds(j * L, L)] = jnp.zeros((L,), jnp.float32)
            return 0

        lax.fori_loop(0, CB, zrow, 0)
        for off, sz in zchunks:
            r0 = tid * tile_rows + off
            if sz > CB:
                pltpu.sync_copy(rowst0, acc.at[pl.ds(r0, CB)])
                pltpu.sync_copy(qv0.at[pl.ds(0, sz - CB)],
                                acc.at[pl.ds(r0 + CB, sz - CB)])
            else:
                pltpu.sync_copy(rowst0.at[pl.ds(0, sz)], acc.at[pl.ds(r0, sz)])
        plsc.subcore_barrier()

        pltpu.sync_copy(st_hbm, st_v)
        s1 = [st_v[0, pl.ds(j * L, L)] for j in range(nj)]
        t1 = [st_v[1, pl.ds(j * L, L)] for j in range(nj)]

        def idx_copy(b, k):
            pltpu.async_copy(src_hbm.at[pl.ds(base + k * CB, CB)], idxs[b], semi[b])

        def idx_wait(b, k):
            pltpu.make_async_copy(
                src_hbm.at[pl.ds(base + k * CB, CB)], idxs[b], semi[b]).wait()

        def idxd_copy(b, k):
            pltpu.async_copy(dst_hbm.at[pl.ds(base + k * CB, CB)], idxd[b], semd[b])

        def idxd_wait(b, k):
            pltpu.make_async_copy(
                dst_hbm.at[pl.ds(base + k * CB, CB)], idxd[b], semd[b]).wait()

        def gath_start(b, k):
            pltpu.async_copy(t_hbm.at[idxs[b]], rowst[b], semg[b])
            pltpu.async_copy(q_hbm.at[pl.ds(base + k * CB, CB)], qvs[b], semg[b])

        def gath_wait(b, k):
            pltpu.make_async_copy(t_hbm.at[idxs[b]], rowst[b], semg[b]).wait()
            pltpu.make_async_copy(
                q_hbm.at[pl.ds(base + k * CB, CB)], qvs[b], semg[b]).wait()

        def scat_start(b):
            pltpu.async_copy(qvs[b], acc.at[idxd[b]], sems[b], add=True)

        def scat_wait(b):
            pltpu.make_async_copy(qvs[b], acc.at[pl.ds(0, CB)], sems[b]).wait()

        def rowloop(b):
            def body(r, _):
                for j in range(nj):
                    sl = pl.ds(j * L, L)
                    y = qvs[b][r, sl] * s1[j] + t1[j] + rowst[b][r, sl]
                    qvs[b][r, sl] = jnp.maximum(y, 0.0)
                return 0

            lax.fori_loop(0, CB, body, 0)

        def phase(b, ck, nxt1, nxt2, first):
            idx_wait(1 - b, nxt1)
            if not first:
                scat_wait(1 - b)
            idxd_copy(1 - b, nxt1)
            gath_start(1 - b, nxt1)
            gath_wait(b, ck)
            idx_copy(b, nxt2)
            rowloop(b)
            idxd_wait(b, ck)
            scat_start(b)

        idx_copy(0, 0)
        idx_wait(0, 0)
        idxd_copy(0, 0)
        gath_start(0, 0)
        idx_copy(1, 1)
        phase(0, 0, 1, 2, True)

        def steady(k2, _):
            c = 2 * k2 + 1
            phase(1, c, c + 1, c + 2, False)
            phase(0, c + 1, c + 2, c + 3, False)
            return 0

        lax.fori_loop(0, (nch - 4) // 2, steady, 0)

        phase(1, nch - 3, nch - 2, nch - 1, False)
        phase(0, nch - 2, nch - 1, nch - 1, False)
        phase(1, nch - 1, nch - 1, nch - 1, False)

        gath_wait(0, nch - 1)
        idx_wait(1, nch - 1)
        idxd_wait(0, nch - 1)
        scat_wait(1)

        if tail:
            pltpu.sync_copy(src_hbm.at[pl.ds(base + nch * CB, tail)], idx_t)
            pltpu.async_copy(t_hbm.at[idx_t], rows_t, semg0).wait()
            pltpu.sync_copy(q_hbm.at[pl.ds(base + nch * CB, tail)], q_t)

            def tbody(r, _):
                for j in range(nj):
                    sl = pl.ds(j * L, L)
                    y = q_t[r, sl] * s1[j] + t1[j] + rows_t[r, sl]
                    q_t[r, sl] = jnp.maximum(y, 0.0)
                return 0

            lax.fori_loop(0, tail, tbody, 0)
            pltpu.sync_copy(dst_hbm.at[pl.ds(base + nch * CB, tail)], idx_t)
            pltpu.sync_copy(q_t, acc.at[idx_t], add=True)

        plsc.subcore_barrier()

        for off, sz in zchunks:
            r0 = tid * tile_rows + off
            if sz > CB:
                pltpu.sync_copy(acc.at[pl.ds(r0, CB)], rowst0)
                pltpu.sync_copy(rowst0, out_hbm.at[sc, pl.ds(r0, CB)])
                pltpu.sync_copy(acc.at[pl.ds(r0 + CB, sz - CB)],
                                qv0.at[pl.ds(0, sz - CB)])
                pltpu.sync_copy(qv0.at[pl.ds(0, sz - CB)],
                                out_hbm.at[sc, pl.ds(r0 + CB, sz - CB)])
            else:
                pltpu.sync_copy(acc.at[pl.ds(r0, sz)], rowst0.at[pl.ds(0, sz)])
                pltpu.sync_copy(rowst0.at[pl.ds(0, sz)],
                                out_hbm.at[sc, pl.ds(r0, sz)])

    return pass_b


def kernel(features, x_atom, edge_attr, edge_index, W1, b1, g1, be1, W2, b2, g2, be2):
    n, h = features.shape
    a = x_atom.shape[1]
    e, bf = edge_attr.shape

    src = edge_index[0]
    dst = edge_index[1]
    w1h = W1[:h]
    w1x = W1[h:h + a]
    w1w = W1[h + a:]

    bn = 1000
    p = pl.pallas_call(
        _a1_body,
        grid=(n // bn,),
        in_specs=[
            pl.BlockSpec((bn, h), lambda i: (i, 0)),
            pl.BlockSpec((bn, a), lambda i: (i, 0)),
            pl.BlockSpec((h, h), lambda i: (0, 0)),
            pl.BlockSpec((a, h), lambda i: (0, 0)),
        ],
        out_specs=pl.BlockSpec((bn, h), lambda i: (i, 0)),
        out_shape=jax.ShapeDtypeStruct((n, h), jnp.float32),
    )(features, x_atom, w1h, w1x)

    be_ = 6400
    q = pl.pallas_call(
        _a2_body,
        grid=(e // be_,),
        in_specs=[
            pl.BlockSpec((bf, be_), lambda i: (0, i)),
            pl.BlockSpec((bf, h), lambda i: (0, 0)),
            pl.BlockSpec((1, h), lambda i: (0, 0)),
        ],
        out_specs=pl.BlockSpec((be_, h), lambda i: (i, 0)),
        out_shape=jax.ShapeDtypeStruct((e, h), jnp.float32),
    )(edge_attr.T, w1w, b1.reshape(1, h))

    parts = _make_pass_a(e, h)(src, p, q).reshape(NW, 2 * h)

    t_tab, st1 = pl.pallas_call(
        functools.partial(_t_body, e),
        grid=(n // bn,),
        in_specs=[
            pl.BlockSpec((NW, 2 * h), lambda i: (0, 0)),
            pl.BlockSpec((1, h), lambda i: (0, 0)),
            pl.BlockSpec((1, h), lambda i: (0, 0)),
            pl.BlockSpec((bn, h), lambda i: (i, 0)),
            pl.BlockSpec((bn, h), lambda i: (i, 0)),
        ],
        out_specs=[
            pl.BlockSpec((bn, h), lambda i: (i, 0)),
            pl.BlockSpec((2, h), lambda i: (0, 0)),
        ],
        out_shape=[
            jax.ShapeDtypeStruct((n, h), jnp.float32),
            jax.ShapeDtypeStruct((2, h), jnp.float32),
        ],
    )(parts, g1.reshape(1, h), be1.reshape(1, h), p, features)

    acc_rows = 10112
    hsc = _make_pass_b(e, n, h)(src, dst, t_tab, q, st1)
    h2 = hsc.reshape(NC * acc_rows, h)

    bs1 = 632
    np1 = acc_rows // bs1
    bs2 = 1000
    np2 = n // bs2
    out = pl.pallas_call(
        functools.partial(_e_body, n, bs1, np1, bs2),
        grid=(np1 + np2,),
        in_specs=[
            pl.BlockSpec((bs1, h), lambda i: (jnp.minimum(i, np1 - 1), 0)),
            pl.BlockSpec((bs1, h), lambda i: (jnp.minimum(i, np1 - 1) + np1, 0)),
            pl.BlockSpec((h, h), lambda i: (0, 0)),
            pl.BlockSpec((1, h), lambda i: (0, 0)),
            pl.BlockSpec((1, h), lambda i: (0, 0)),
            pl.BlockSpec((1, h), lambda i: (0, 0)),
            pl.BlockSpec((bs2, h), lambda i: (jnp.maximum(i - np1, 0), 0)),
        ],
        out_specs=pl.BlockSpec((bs2, h), lambda i: (jnp.maximum(i - np1, 0), 0)),
        out_shape=jax.ShapeDtypeStruct((n, h), jnp.float32),
        scratch_shapes=[
            pltpu.VMEM((acc_rows, h), jnp.float32),
            pltpu.VMEM((2, h), jnp.float32),
            pltpu.VMEM((2, h), jnp.float32),
        ],
    )(h2, h2, W2, b2.reshape(1, h), g2.reshape(1, h), be2.reshape(1, h),
      features)
    return out
